# Initial kernel scaffold; baseline (speedup 1.0000x reference)
import jax
import jax.numpy as jnp
from jax.experimental import pallas as pl


def kernel(x, table):
    raise NotImplementedError("write your pallas kernel here")


# speedup vs baseline: 16.5850x; 16.5850x over previous
import functools

import jax
import jax.numpy as jnp
from jax import lax
from jax.experimental import pallas as pl
from jax.experimental.pallas import tpu as pltpu
from jax.experimental.pallas import tpu_sc as plsc


def kernel(x, table):
    N, S, H, W, D = x.shape
    V, E = table.shape
    P = H * W * D
    L = 16
    NC, NS = 2, 16
    NW = NC * NS

    assert S % NW == 0, (S, NW)
    SPW = S // NW
    CHUNKS = 4
    assert P % CHUNKS == 0
    R = P // CHUNKS
    assert R % 8 == 0 and P % 8 == 0 and E % L == 0
    assert 2 * R * E * 4 <= 524284

    mesh = plsc.VectorSubcoreMesh(core_axis_name="c", subcore_axis_name="s")

    @functools.partial(
        pl.kernel,
        out_type=jax.ShapeDtypeStruct((N * S * P * E,), jnp.float32),
        name="positional_embedding_broadcast",
        mesh=mesh,
        scratch_types=[
            pltpu.VMEM((2, R * E), jnp.float32),
            pltpu.SemaphoreType.DMA,
        ],
    )
    def emb(table_hbm, out_hbm, buf, sem):
        wid = lax.axis_index("s") * NC + lax.axis_index("c")
        pending = []
        for si in range(SPW):
            s = wid * SPW + si
            bslot = buf.at[si % 2]
            pltpu.sync_copy(table_hbm.at[pl.ds(s * E, E)], bslot.at[pl.ds(0, E)])
            row = [bslot[pl.ds(k * L, L)] for k in range(E // L)]

            def fill(i, _):
                for k in range(E // L):
                    bslot[pl.ds(i * E + k * L, L)] = row[k]
                return 0

            lax.fori_loop(1, R, fill, 0)

            for cp in pending:
                cp.wait()
            pending = []
            for n in range(N):
                for c in range(CHUNKS):
                    dst = out_hbm.at[pl.ds(((n * S + s) * P + c * R) * E, R * E)]
                    pending.append(pltpu.async_copy(bslot, dst, sem))
        for cp in pending:
            cp.wait()

    out = emb(table.reshape(-1))
    return out.reshape(N, S, H, W, D, E)
